# baseline (speedup 1.0000x reference)
import functools

import jax
import jax.numpy as jnp
import numpy as np
from jax import lax
from jax.experimental import pallas as pl
from jax.experimental.pallas import tpu as pltpu
from jax.experimental.pallas import tpu_sc as plsc

Q_MIN = 1.0
K_MAX = 256
KPAD = 272
N_LANES = 16
N_SUBCORES = 16
N_CORES = 2


def _sc_stats_attrep(n_ev, c_q):
    chunk = -(-n_ev // (N_SUBCORES * N_LANES)) * N_LANES
    g_full = chunk // N_LANES
    last_size = n_ev - (N_SUBCORES - 1) * chunk
    assert last_size % N_LANES == 0 and last_size > 0
    g_last = last_size // N_LANES
    mesh = plsc.VectorSubcoreMesh(core_axis_name="c", subcore_axis_name="s")

    @functools.partial(
        pl.kernel,
        mesh=mesh,
        out_type=jax.ShapeDtypeStruct((N_CORES * 6 * KPAD,), jnp.float32),
        compiler_params=pltpu.CompilerParams(needs_layout_passes=False),
        scratch_types=[
            pltpu.VMEM((chunk,), jnp.int32),
            pltpu.VMEM((3 * chunk,), jnp.float32),
            pltpu.VMEM((N_LANES * KPAD,), jnp.float32),
            pltpu.VMEM((N_LANES * KPAD,), jnp.float32),
            pltpu.VMEM((N_LANES * KPAD,), jnp.float32),
            pltpu.VMEM((N_LANES * KPAD,), jnp.float32),
            pltpu.VMEM((4 * KPAD,), jnp.float32),
            pltpu.VMEM((2 * KPAD,), jnp.float32),
            pltpu.VMEM((3 * KPAD,), jnp.float32),
            pltpu.VMEM_SHARED((N_SUBCORES * 4 * KPAD,), jnp.float32),
            pltpu.VMEM((N_SUBCORES * 4 * KPAD,), jnp.float32),
        ],
    )
    def sc_kernel(tidx_hbm, coords_hbm, out_hbm,
                  idx_v, crd_v, acc0, acc1, acc2, acc3, red4_v, red2_v,
                  ctr_v, shared, gath_v):
        c = lax.axis_index("c")
        s = lax.axis_index("s")

        base = c * n_ev + s * chunk
        is_last = s == N_SUBCORES - 1

        @pl.when(is_last)
        def _():
            pltpu.sync_copy(tidx_hbm.at[pl.ds(base, last_size)],
                            idx_v.at[pl.ds(0, last_size)])
            pltpu.sync_copy(coords_hbm.at[pl.ds(3 * base, 3 * last_size)],
                            crd_v.at[pl.ds(0, 3 * last_size)])

        @pl.when(jnp.logical_not(is_last))
        def _():
            pltpu.sync_copy(tidx_hbm.at[pl.ds(base, chunk)], idx_v)
            pltpu.sync_copy(coords_hbm.at[pl.ds(3 * base, 3 * chunk)], crd_v)

        groups = jnp.where(is_last, g_last, g_full)

        zeros16 = jnp.zeros((N_LANES,), jnp.float32)
        lane_off = lax.iota(jnp.int32, N_LANES) * KPAD
        iota3 = lax.iota(jnp.int32, N_LANES) * 3
        ones16 = jnp.ones((N_LANES,), jnp.float32)

        def zero4_body(j, carry):
            sl = pl.ds(j * N_LANES, N_LANES)
            acc0[sl] = zeros16
            acc1[sl] = zeros16
            acc2[sl] = zeros16
            acc3[sl] = zeros16
            return carry

        lax.fori_loop(0, KPAD, zero4_body, 0)

        def scat_body(g, carry):
            sl = pl.ds(g * N_LANES, N_LANES)
            fidx = idx_v[sl] + lane_off
            gx = g * (3 * N_LANES) + iota3
            plsc.addupdate_scatter(acc0, [fidx], ones16)
            plsc.addupdate_scatter(acc1, [fidx], plsc.load_gather(crd_v, [gx]))
            plsc.addupdate_scatter(acc2, [fidx], plsc.load_gather(crd_v, [gx + 1]))
            plsc.addupdate_scatter(acc3, [fidx], plsc.load_gather(crd_v, [gx + 2]))
            return carry

        lax.fori_loop(0, groups, scat_body, 0)

        def lred4_body(j, carry):
            for q, acc in enumerate((acc0, acc1, acc2, acc3)):
                v = acc[pl.ds(j * N_LANES, N_LANES)]
                for l in range(1, N_LANES):
                    v = v + acc[pl.ds(l * KPAD + j * N_LANES, N_LANES)]
                red4_v[pl.ds(q * KPAD + j * N_LANES, N_LANES)] = v
            return carry

        lax.fori_loop(0, KPAD // N_LANES, lred4_body, 0)

        pltpu.sync_copy(red4_v, shared.at[pl.ds(s * 4 * KPAD, 4 * KPAD)])
        plsc.subcore_barrier()

        @pl.when(s == 0)
        def _():
            pltpu.sync_copy(shared, gath_v)

            def tred4_body(j, carry):
                for q in range(4):
                    off = q * KPAD + j * N_LANES
                    v = gath_v[pl.ds(off, N_LANES)]
                    for t in range(1, N_SUBCORES):
                        v = v + gath_v[pl.ds(t * 4 * KPAD + off, N_LANES)]
                    red4_v[pl.ds(off, N_LANES)] = v
                return carry

            lax.fori_loop(0, KPAD // N_LANES, tred4_body, 0)
            pltpu.sync_copy(red4_v, out_hbm.at[pl.ds(c * 6 * KPAD, 4 * KPAD)])
            pltpu.sync_copy(red4_v, shared.at[pl.ds(0, 4 * KPAD)])

        plsc.subcore_barrier()

        pltpu.sync_copy(shared.at[pl.ds(0, 4 * KPAD)], red4_v)
        plsc.subcore_barrier()

        def ctr_body(j, carry):
            sl = pl.ds(j * N_LANES, N_LANES)
            nk = red4_v[sl]
            inv = c_q / jnp.maximum(nk * c_q, 1e-6)
            for d in range(3):
                ctr_v[pl.ds(d * KPAD + j * N_LANES, N_LANES)] = (
                    red4_v[pl.ds((1 + d) * KPAD + j * N_LANES, N_LANES)] * inv)
            return carry

        lax.fori_loop(0, KPAD // N_LANES, ctr_body, 0)

        def zero2_body(j, carry):
            sl = pl.ds(j * N_LANES, N_LANES)
            acc0[sl] = zeros16
            acc1[sl] = zeros16
            return carry

        lax.fori_loop(0, KPAD, zero2_body, 0)

        magic = jnp.full((N_LANES,), 0x5F3759DF, jnp.int32)

        def hit_body(g, carry):
            sl = pl.ds(g * N_LANES, N_LANES)
            ti = idx_v[sl]
            gx = g * (3 * N_LANES) + iota3
            dx = plsc.load_gather(crd_v, [gx]) - plsc.load_gather(ctr_v, [ti])
            dy = plsc.load_gather(crd_v, [gx + 1]) - plsc.load_gather(
                ctr_v, [ti + KPAD])
            dz = plsc.load_gather(crd_v, [gx + 2]) - plsc.load_gather(
                ctr_v, [ti + 2 * KPAD])
            d2 = dx * dx + dy * dy + dz * dz
            fidx = ti + lane_off
            plsc.addupdate_scatter(acc0, [fidx], d2)
            t = d2 + 1e-9
            th = t * 0.5
            y = plsc.bitcast(magic - (plsc.bitcast(t, jnp.int32) >> 1),
                             jnp.float32)
            y = y * (1.5 - th * y * y)
            y = y * (1.5 - th * y * y)
            hinge = jnp.maximum(1.0 - t * y, 0.0)
            plsc.addupdate_scatter(acc1, [fidx], hinge)
            return carry

        lax.fori_loop(0, groups, hit_body, 0)

        def lred2_body(j, carry):
            for q, acc in enumerate((acc0, acc1)):
                v = acc[pl.ds(j * N_LANES, N_LANES)]
                for l in range(1, N_LANES):
                    v = v + acc[pl.ds(l * KPAD + j * N_LANES, N_LANES)]
                red2_v[pl.ds(q * KPAD + j * N_LANES, N_LANES)] = v
            return carry

        lax.fori_loop(0, KPAD // N_LANES, lred2_body, 0)

        pltpu.sync_copy(red2_v, shared.at[pl.ds(s * 2 * KPAD, 2 * KPAD)])
        plsc.subcore_barrier()

        @pl.when(s == 0)
        def _():
            pltpu.sync_copy(shared.at[pl.ds(0, N_SUBCORES * 2 * KPAD)],
                            gath_v.at[pl.ds(0, N_SUBCORES * 2 * KPAD)])

            def tred2_body(j, carry):
                for q in range(2):
                    off = q * KPAD + j * N_LANES
                    v = gath_v[pl.ds(off, N_LANES)]
                    for t in range(1, N_SUBCORES):
                        v = v + gath_v[pl.ds(t * 2 * KPAD + off, N_LANES)]
                    red2_v[pl.ds(off, N_LANES)] = v
                return carry

            lax.fori_loop(0, KPAD // N_LANES, tred2_body, 0)
            pltpu.sync_copy(red2_v,
                            out_hbm.at[pl.ds(c * 6 * KPAD + 4 * KPAD, 2 * KPAD)])

    return sc_kernel


def _tc_dense(coords_ref, stats_ref, out_ref, *, tiles_per_event, c_q):
    i = pl.program_id(0)
    t = lax.rem(i, tiles_per_event)

    nk = stats_ref[0, 0, :K_MAX]
    inv = c_q / jnp.maximum(nk * c_q, 1e-6)
    mx = stats_ref[0, 1, :K_MAX] * inv
    my = stats_ref[0, 2, :K_MAX] * inv
    mz = stats_ref[0, 3, :K_MAX] * inv
    mn = mx * mx + my * my + mz * mz
    m5 = jnp.stack([-2.0 * mx, -2.0 * my, -2.0 * mz, mn,
                    jnp.ones_like(mn)])
    ce = coords_ref[...]
    cn = jnp.sum(ce * ce, axis=1, keepdims=True) + 1e-9
    ce5 = jnp.concatenate([ce, jnp.ones_like(cn), cn], axis=1)
    tt = lax.dot_general(ce5, m5, (((1,), (0,)), ((), ())),
                         preferred_element_type=jnp.float32)
    tt = jnp.maximum(tt, 1e-9)
    hinge = jnp.maximum(1.0 - tt * lax.rsqrt(tt), 0.0)
    part = jnp.sum(hinge, axis=0)[None, None]

    @pl.when(t == 0)
    def _():
        out_ref[...] = part

    @pl.when(t != 0)
    def _():
        out_ref[...] = out_ref[...] + part


def kernel(x, predCCoords, truthHitAssignementIdx, row_splits):
    del x, row_splits
    coords = predCCoords.astype(jnp.float32)
    tidx = truthHitAssignementIdx.reshape(-1).astype(jnp.int32)
    n = coords.shape[0]
    n_ev = n // 2
    c_q = float(np.arctanh(0.5) ** 2 + Q_MIN)

    sc_out = _sc_stats_attrep(n_ev, c_q)(
        tidx, coords.reshape(-1)).reshape(2, 6, KPAD)
    stats = sc_out[:, :4, :]

    tile_n = 10000
    tiles_per_event = n_ev // tile_n
    grid = 2 * tiles_per_event

    repall = pl.pallas_call(
        functools.partial(_tc_dense, tiles_per_event=tiles_per_event,
                          c_q=c_q),
        grid=(grid,),
        in_specs=[
            pl.BlockSpec((tile_n, 3), lambda i: (i, 0)),
            pl.BlockSpec((1, 4, KPAD),
                         lambda i: (i // (grid // 2), 0, 0)),
        ],
        out_specs=pl.BlockSpec((1, 1, K_MAX),
                               lambda i: (i // (grid // 2), 0, 0)),
        out_shape=jax.ShapeDtypeStruct((2, 1, K_MAX), jnp.float32),
    )(coords, stats)[:, 0, :]

    nk = stats[:, 0, :K_MAX]
    att_s = sc_out[:, 4, :K_MAX]
    rep_o = sc_out[:, 5, :K_MAX]
    exists = (nk > 0).astype(jnp.float32)
    c2 = jnp.float32(c_q * c_q)
    att = c2 * att_s / jnp.maximum(nk, 1.0)
    rep = c2 * (repall - rep_o) / jnp.maximum(float(n_ev) - nk, 1.0)
    n_obj = jnp.maximum(jnp.sum(exists, axis=1), 1.0)
    v_att = jnp.sum(att * exists, axis=1) / n_obj
    v_rep = jnp.sum(rep * exists, axis=1) / n_obj
    return jnp.sum(v_att + v_rep) / 2.0


# speedup vs baseline: 1.5380x; 1.5380x over previous
import functools

import jax
import jax.numpy as jnp
import numpy as np
from jax import lax
from jax.experimental import pallas as pl
from jax.experimental.pallas import tpu as pltpu
from jax.experimental.pallas import tpu_sc as plsc

Q_MIN = 1.0
K_MAX = 256
KPAD = 272
N_LANES = 16
N_SUBCORES = 16
N_CORES = 2


def _sc_stats_attrep(nep, chunk, c_q):
    groups = chunk // N_LANES
    mesh = plsc.VectorSubcoreMesh(core_axis_name="c", subcore_axis_name="s")

    @functools.partial(
        pl.kernel,
        mesh=mesh,
        out_type=jax.ShapeDtypeStruct((N_CORES * 6 * KPAD,), jnp.float32),
        compiler_params=pltpu.CompilerParams(needs_layout_passes=False),
        scratch_types=[
            pltpu.VMEM((chunk,), jnp.int32),
            pltpu.VMEM((3 * chunk,), jnp.float32),
            pltpu.VMEM((N_LANES * KPAD,), jnp.float32),
            pltpu.VMEM((N_LANES * KPAD,), jnp.float32),
            pltpu.VMEM((N_LANES * KPAD,), jnp.float32),
            pltpu.VMEM((N_LANES * KPAD,), jnp.float32),
            pltpu.VMEM((4 * KPAD,), jnp.float32),
            pltpu.VMEM((2 * KPAD,), jnp.float32),
            pltpu.VMEM((3 * KPAD,), jnp.float32),
            pltpu.VMEM_SHARED((N_SUBCORES * 4 * KPAD,), jnp.float32),
            pltpu.VMEM((N_SUBCORES * 4 * KPAD,), jnp.float32),
        ],
    )
    def sc_kernel(tidx_hbm, coords_hbm, out_hbm,
                  idx_v, crd_v, acc0, acc1, acc2, acc3, red4_v, red2_v,
                  ctr_v, shared, gath_v):
        c = lax.axis_index("c")
        s = lax.axis_index("s")

        base = c * nep + s * chunk
        pltpu.sync_copy(tidx_hbm.at[pl.ds(base, chunk)], idx_v)
        for d in range(3):
            pltpu.sync_copy(
                coords_hbm.at[pl.ds((c * 3 + d) * nep + s * chunk, chunk)],
                crd_v.at[pl.ds(d * chunk, chunk)])

        zeros16 = jnp.zeros((N_LANES,), jnp.float32)
        lane_off = lax.iota(jnp.int32, N_LANES) * KPAD
        ones16 = jnp.ones((N_LANES,), jnp.float32)

        def zero4_body(j, carry):
            sl = pl.ds(j * N_LANES, N_LANES)
            acc0[sl] = zeros16
            acc1[sl] = zeros16
            acc2[sl] = zeros16
            acc3[sl] = zeros16
            return carry

        lax.fori_loop(0, KPAD, zero4_body, 0)

        def scat_body(g, carry):
            sl = pl.ds(g * N_LANES, N_LANES)
            fidx = idx_v[sl] + lane_off
            plsc.addupdate_scatter(acc0, [fidx], ones16)
            plsc.addupdate_scatter(acc1, [fidx], crd_v[pl.ds(g * N_LANES, N_LANES)])
            plsc.addupdate_scatter(acc2, [fidx], crd_v[pl.ds(chunk + g * N_LANES, N_LANES)])
            plsc.addupdate_scatter(acc3, [fidx], crd_v[pl.ds(2 * chunk + g * N_LANES, N_LANES)])
            return carry

        lax.fori_loop(0, groups, scat_body, 0)

        def lred4_body(j, carry):
            for q, acc in enumerate((acc0, acc1, acc2, acc3)):
                v = acc[pl.ds(j * N_LANES, N_LANES)]
                for l in range(1, N_LANES):
                    v = v + acc[pl.ds(l * KPAD + j * N_LANES, N_LANES)]
                red4_v[pl.ds(q * KPAD + j * N_LANES, N_LANES)] = v
            return carry

        lax.fori_loop(0, KPAD // N_LANES, lred4_body, 0)

        pltpu.sync_copy(red4_v, shared.at[pl.ds(s * 4 * KPAD, 4 * KPAD)])
        plsc.subcore_barrier()

        @pl.when(s == 0)
        def _():
            pltpu.sync_copy(shared, gath_v)

            def tred4_body(j, carry):
                for q in range(4):
                    off = q * KPAD + j * N_LANES
                    v = gath_v[pl.ds(off, N_LANES)]
                    for t in range(1, N_SUBCORES):
                        v = v + gath_v[pl.ds(t * 4 * KPAD + off, N_LANES)]
                    red4_v[pl.ds(off, N_LANES)] = v
                return carry

            lax.fori_loop(0, KPAD // N_LANES, tred4_body, 0)
            pltpu.sync_copy(red4_v, out_hbm.at[pl.ds(c * 6 * KPAD, 4 * KPAD)])
            pltpu.sync_copy(red4_v, shared.at[pl.ds(0, 4 * KPAD)])

        plsc.subcore_barrier()

        pltpu.sync_copy(shared.at[pl.ds(0, 4 * KPAD)], red4_v)
        plsc.subcore_barrier()

        def ctr_body(j, carry):
            sl = pl.ds(j * N_LANES, N_LANES)
            nk = red4_v[sl]
            inv = c_q / jnp.maximum(nk * c_q, 1e-6)
            for d in range(3):
                ctr_v[pl.ds(d * KPAD + j * N_LANES, N_LANES)] = (
                    red4_v[pl.ds((1 + d) * KPAD + j * N_LANES, N_LANES)] * inv)
            return carry

        lax.fori_loop(0, KPAD // N_LANES, ctr_body, 0)

        def zero2_body(j, carry):
            sl = pl.ds(j * N_LANES, N_LANES)
            acc0[sl] = zeros16
            acc1[sl] = zeros16
            return carry

        lax.fori_loop(0, KPAD, zero2_body, 0)

        magic = jnp.full((N_LANES,), 0x5F3759DF, jnp.int32)

        def hit_body(g, carry):
            sl = pl.ds(g * N_LANES, N_LANES)
            ti = idx_v[sl]
            dx = crd_v[pl.ds(g * N_LANES, N_LANES)] - plsc.load_gather(ctr_v, [ti])
            dy = crd_v[pl.ds(chunk + g * N_LANES, N_LANES)] - plsc.load_gather(
                ctr_v, [ti + KPAD])
            dz = crd_v[pl.ds(2 * chunk + g * N_LANES, N_LANES)] - plsc.load_gather(
                ctr_v, [ti + 2 * KPAD])
            d2 = dx * dx + dy * dy + dz * dz
            fidx = ti + lane_off
            plsc.addupdate_scatter(acc0, [fidx], d2)
            t = d2 + 1e-9
            th = t * 0.5
            y = plsc.bitcast(magic - (plsc.bitcast(t, jnp.int32) >> 1),
                             jnp.float32)
            y = y * (1.5 - th * y * y)
            y = y * (1.5 - th * y * y)
            hinge = jnp.maximum(1.0 - t * y, 0.0)
            plsc.addupdate_scatter(acc1, [fidx], hinge)
            return carry

        lax.fori_loop(0, groups, hit_body, 0)

        def lred2_body(j, carry):
            for q, acc in enumerate((acc0, acc1)):
                v = acc[pl.ds(j * N_LANES, N_LANES)]
                for l in range(1, N_LANES):
                    v = v + acc[pl.ds(l * KPAD + j * N_LANES, N_LANES)]
                red2_v[pl.ds(q * KPAD + j * N_LANES, N_LANES)] = v
            return carry

        lax.fori_loop(0, KPAD // N_LANES, lred2_body, 0)

        pltpu.sync_copy(red2_v, shared.at[pl.ds(s * 2 * KPAD, 2 * KPAD)])
        plsc.subcore_barrier()

        @pl.when(s == 0)
        def _():
            pltpu.sync_copy(shared.at[pl.ds(0, N_SUBCORES * 2 * KPAD)],
                            gath_v.at[pl.ds(0, N_SUBCORES * 2 * KPAD)])

            def tred2_body(j, carry):
                for q in range(2):
                    off = q * KPAD + j * N_LANES
                    v = gath_v[pl.ds(off, N_LANES)]
                    for t in range(1, N_SUBCORES):
                        v = v + gath_v[pl.ds(t * 2 * KPAD + off, N_LANES)]
                    red2_v[pl.ds(off, N_LANES)] = v
                return carry

            lax.fori_loop(0, KPAD // N_LANES, tred2_body, 0)
            pltpu.sync_copy(red2_v,
                            out_hbm.at[pl.ds(c * 6 * KPAD + 4 * KPAD, 2 * KPAD)])

    return sc_kernel


def _tc_dense(coords_ref, stats_ref, out_ref, *, tiles_per_event, c_q):
    i = pl.program_id(0)
    t = lax.rem(i, tiles_per_event)

    nk = stats_ref[0, 0, :K_MAX]
    inv = c_q / jnp.maximum(nk * c_q, 1e-6)
    mx = stats_ref[0, 1, :K_MAX] * inv
    my = stats_ref[0, 2, :K_MAX] * inv
    mz = stats_ref[0, 3, :K_MAX] * inv
    mn = mx * mx + my * my + mz * mz
    m5 = jnp.stack([-2.0 * mx, -2.0 * my, -2.0 * mz, mn,
                    jnp.ones_like(mn)])
    ce = coords_ref[...]
    cn = jnp.sum(ce * ce, axis=1, keepdims=True) + 1e-9
    ce5 = jnp.concatenate([ce, jnp.ones_like(cn), cn], axis=1)
    tt = lax.dot_general(ce5, m5, (((1,), (0,)), ((), ())),
                         preferred_element_type=jnp.float32)
    tt = jnp.maximum(tt, 1e-9)
    hinge = jnp.maximum(1.0 - tt * lax.rsqrt(tt), 0.0)
    part = jnp.sum(hinge, axis=0)[None, None]

    @pl.when(t == 0)
    def _():
        out_ref[...] = part

    @pl.when(t != 0)
    def _():
        out_ref[...] = out_ref[...] + part


def kernel(x, predCCoords, truthHitAssignementIdx, row_splits):
    del x, row_splits
    coords = predCCoords.astype(jnp.float32)
    tidx = truthHitAssignementIdx.reshape(-1).astype(jnp.int32)
    n = coords.shape[0]
    n_ev = n // 2
    c_q = float(np.arctanh(0.5) ** 2 + Q_MIN)

    chunk = -(-n_ev // (N_SUBCORES * N_LANES)) * N_LANES
    nep = chunk * N_SUBCORES
    pad = nep - n_ev
    t0 = jnp.concatenate([tidx[:n_ev], jnp.full((pad,), K_MAX, jnp.int32)])
    t1 = jnp.concatenate([tidx[n_ev:], jnp.full((pad,), K_MAX, jnp.int32)])
    tidx_flat = jnp.concatenate([t0, t1])
    cpad = jnp.pad(coords.T.reshape(3, 2, n_ev), ((0, 0), (0, 0), (0, pad)))
    coords_flat = cpad.transpose(1, 0, 2).reshape(-1)

    sc_out = _sc_stats_attrep(nep, chunk, c_q)(
        tidx_flat, coords_flat).reshape(2, 6, KPAD)
    stats = sc_out[:, :4, :]

    tile_n = 10000
    tiles_per_event = n_ev // tile_n
    grid = 2 * tiles_per_event

    repall = pl.pallas_call(
        functools.partial(_tc_dense, tiles_per_event=tiles_per_event,
                          c_q=c_q),
        grid=(grid,),
        in_specs=[
            pl.BlockSpec((tile_n, 3), lambda i: (i, 0)),
            pl.BlockSpec((1, 4, KPAD),
                         lambda i: (i // (grid // 2), 0, 0)),
        ],
        out_specs=pl.BlockSpec((1, 1, K_MAX),
                               lambda i: (i // (grid // 2), 0, 0)),
        out_shape=jax.ShapeDtypeStruct((2, 1, K_MAX), jnp.float32),
    )(coords, stats)[:, 0, :]

    nk = stats[:, 0, :K_MAX]
    att_s = sc_out[:, 4, :K_MAX]
    rep_o = sc_out[:, 5, :K_MAX]
    exists = (nk > 0).astype(jnp.float32)
    c2 = jnp.float32(c_q * c_q)
    att = c2 * att_s / jnp.maximum(nk, 1.0)
    rep = c2 * (repall - rep_o) / jnp.maximum(float(n_ev) - nk, 1.0)
    n_obj = jnp.maximum(jnp.sum(exists, axis=1), 1.0)
    v_att = jnp.sum(att * exists, axis=1) / n_obj
    v_rep = jnp.sum(rep * exists, axis=1) / n_obj
    return jnp.sum(v_att + v_rep) / 2.0


# speedup vs baseline: 1.7147x; 1.1148x over previous
import functools

import jax
import jax.numpy as jnp
import numpy as np
from jax import lax
from jax.experimental import pallas as pl
from jax.experimental.pallas import tpu as pltpu
from jax.experimental.pallas import tpu_sc as plsc

Q_MIN = 1.0
K_MAX = 256
KPAD = 272
N_LANES = 16
N_SUBCORES = 16
N_CORES = 2


def _sc_stats_attrep(nep, chunk, c_q):
    groups = chunk // N_LANES
    mesh = plsc.VectorSubcoreMesh(core_axis_name="c", subcore_axis_name="s")

    @functools.partial(
        pl.kernel,
        mesh=mesh,
        out_type=jax.ShapeDtypeStruct((N_CORES * 6 * KPAD,), jnp.float32),
        compiler_params=pltpu.CompilerParams(needs_layout_passes=False),
        scratch_types=[
            pltpu.VMEM((chunk,), jnp.int32),
            pltpu.VMEM((3 * chunk,), jnp.float32),
            pltpu.VMEM((N_LANES * KPAD,), jnp.float32),
            pltpu.VMEM((N_LANES * KPAD,), jnp.float32),
            pltpu.VMEM((N_LANES * KPAD,), jnp.float32),
            pltpu.VMEM((N_LANES * KPAD,), jnp.float32),
            pltpu.VMEM((4 * KPAD,), jnp.float32),
            pltpu.VMEM((2 * KPAD,), jnp.float32),
            pltpu.VMEM((3 * KPAD,), jnp.float32),
            pltpu.VMEM_SHARED((N_SUBCORES * 4 * KPAD,), jnp.float32),
            pltpu.VMEM((N_SUBCORES * 4 * KPAD,), jnp.float32),
        ],
    )
    def sc_kernel(tidx_hbm, coords_hbm, out_hbm,
                  idx_v, crd_v, acc0, acc1, acc2, acc3, red4_v, red2_v,
                  ctr_v, shared, gath_v):
        c = lax.axis_index("c")
        s = lax.axis_index("s")

        base = c * nep + s * chunk
        pltpu.sync_copy(tidx_hbm.at[pl.ds(base, chunk)], idx_v)
        for d in range(3):
            pltpu.sync_copy(
                coords_hbm.at[pl.ds((c * 3 + d) * nep + s * chunk, chunk)],
                crd_v.at[pl.ds(d * chunk, chunk)])

        zeros16 = jnp.zeros((N_LANES,), jnp.float32)
        lane_off = lax.iota(jnp.int32, N_LANES) * KPAD
        ones16 = jnp.ones((N_LANES,), jnp.float32)

        def zero4_body(j, carry):
            sl = pl.ds(j * N_LANES, N_LANES)
            acc0[sl] = zeros16
            acc1[sl] = zeros16
            acc2[sl] = zeros16
            acc3[sl] = zeros16
            return carry

        lax.fori_loop(0, KPAD, zero4_body, 0)

        def scat_body(g, carry):
            sl = pl.ds(g * N_LANES, N_LANES)
            fidx = idx_v[sl] + lane_off
            plsc.addupdate_scatter(acc0, [fidx], ones16)
            plsc.addupdate_scatter(acc1, [fidx], crd_v[pl.ds(g * N_LANES, N_LANES)])
            plsc.addupdate_scatter(acc2, [fidx], crd_v[pl.ds(chunk + g * N_LANES, N_LANES)])
            plsc.addupdate_scatter(acc3, [fidx], crd_v[pl.ds(2 * chunk + g * N_LANES, N_LANES)])
            return carry

        lax.fori_loop(0, groups, scat_body, 0)

        def lred4_body(j, carry):
            for q, acc in enumerate((acc0, acc1, acc2, acc3)):
                v = acc[pl.ds(j * N_LANES, N_LANES)]
                for l in range(1, N_LANES):
                    v = v + acc[pl.ds(l * KPAD + j * N_LANES, N_LANES)]
                red4_v[pl.ds(q * KPAD + j * N_LANES, N_LANES)] = v
            return carry

        lax.fori_loop(0, KPAD // N_LANES, lred4_body, 0)

        pltpu.sync_copy(red4_v, shared.at[pl.ds(s * 4 * KPAD, 4 * KPAD)])
        plsc.subcore_barrier()

        @pl.when(s == 0)
        def _():
            pltpu.sync_copy(shared, gath_v)

            def tred4_body(j, carry):
                for q in range(4):
                    off = q * KPAD + j * N_LANES
                    v = gath_v[pl.ds(off, N_LANES)]
                    for t in range(1, N_SUBCORES):
                        v = v + gath_v[pl.ds(t * 4 * KPAD + off, N_LANES)]
                    red4_v[pl.ds(off, N_LANES)] = v
                return carry

            lax.fori_loop(0, KPAD // N_LANES, tred4_body, 0)
            pltpu.sync_copy(red4_v, out_hbm.at[pl.ds(c * 6 * KPAD, 4 * KPAD)])
            pltpu.sync_copy(red4_v, shared.at[pl.ds(0, 4 * KPAD)])

        plsc.subcore_barrier()

        pltpu.sync_copy(shared.at[pl.ds(0, 4 * KPAD)], red4_v)
        plsc.subcore_barrier()

        def ctr_body(j, carry):
            sl = pl.ds(j * N_LANES, N_LANES)
            nk = red4_v[sl]
            inv = c_q / jnp.maximum(nk * c_q, 1e-6)
            for d in range(3):
                ctr_v[pl.ds(d * KPAD + j * N_LANES, N_LANES)] = (
                    red4_v[pl.ds((1 + d) * KPAD + j * N_LANES, N_LANES)] * inv)
            return carry

        lax.fori_loop(0, KPAD // N_LANES, ctr_body, 0)

        def zero2_body(j, carry):
            sl = pl.ds(j * N_LANES, N_LANES)
            acc0[sl] = zeros16
            acc1[sl] = zeros16
            return carry

        lax.fori_loop(0, KPAD, zero2_body, 0)

        magic = jnp.full((N_LANES,), 0x5F3759DF, jnp.int32)

        def hit_body(g, carry):
            sl = pl.ds(g * N_LANES, N_LANES)
            ti = idx_v[sl]
            dx = crd_v[pl.ds(g * N_LANES, N_LANES)] - plsc.load_gather(ctr_v, [ti])
            dy = crd_v[pl.ds(chunk + g * N_LANES, N_LANES)] - plsc.load_gather(
                ctr_v, [ti + KPAD])
            dz = crd_v[pl.ds(2 * chunk + g * N_LANES, N_LANES)] - plsc.load_gather(
                ctr_v, [ti + 2 * KPAD])
            d2 = dx * dx + dy * dy + dz * dz
            fidx = ti + lane_off
            plsc.addupdate_scatter(acc0, [fidx], d2)
            t = d2 + 1e-9
            th = t * 0.5
            y = plsc.bitcast(magic - (plsc.bitcast(t, jnp.int32) >> 1),
                             jnp.float32)
            y = y * (1.5 - th * y * y)
            y = y * (1.5 - th * y * y)
            hinge = jnp.maximum(1.0 - t * y, 0.0)
            plsc.addupdate_scatter(acc1, [fidx], hinge)
            return carry

        lax.fori_loop(0, groups, hit_body, 0)

        def lred2_body(j, carry):
            for q, acc in enumerate((acc0, acc1)):
                v = acc[pl.ds(j * N_LANES, N_LANES)]
                for l in range(1, N_LANES):
                    v = v + acc[pl.ds(l * KPAD + j * N_LANES, N_LANES)]
                red2_v[pl.ds(q * KPAD + j * N_LANES, N_LANES)] = v
            return carry

        lax.fori_loop(0, KPAD // N_LANES, lred2_body, 0)

        pltpu.sync_copy(red2_v, shared.at[pl.ds(s * 2 * KPAD, 2 * KPAD)])
        plsc.subcore_barrier()

        @pl.when(s == 0)
        def _():
            pltpu.sync_copy(shared.at[pl.ds(0, N_SUBCORES * 2 * KPAD)],
                            gath_v.at[pl.ds(0, N_SUBCORES * 2 * KPAD)])

            def tred2_body(j, carry):
                for q in range(2):
                    off = q * KPAD + j * N_LANES
                    v = gath_v[pl.ds(off, N_LANES)]
                    for t in range(1, N_SUBCORES):
                        v = v + gath_v[pl.ds(t * 2 * KPAD + off, N_LANES)]
                    red2_v[pl.ds(off, N_LANES)] = v
                return carry

            lax.fori_loop(0, KPAD // N_LANES, tred2_body, 0)
            pltpu.sync_copy(red2_v,
                            out_hbm.at[pl.ds(c * 6 * KPAD + 4 * KPAD, 2 * KPAD)])

    return sc_kernel


def _tc_dense(coords_ref, stats_ref, out_ref, *, tiles_per_event, c_q):
    i = pl.program_id(0)
    t = lax.rem(i, tiles_per_event)

    nk = stats_ref[0, 0, :K_MAX]
    inv = c_q / jnp.maximum(nk * c_q, 1e-6)
    mx = stats_ref[0, 1, :K_MAX] * inv
    my = stats_ref[0, 2, :K_MAX] * inv
    mz = stats_ref[0, 3, :K_MAX] * inv
    mn = mx * mx + my * my + mz * mz
    m5 = jnp.stack([-2.0 * mx, -2.0 * my, -2.0 * mz, mn,
                    jnp.ones_like(mn)])
    ce = coords_ref[0]
    cn = jnp.sum(ce * ce, axis=0, keepdims=True) + 1e-9
    ce5 = jnp.concatenate([ce, jnp.ones_like(cn), cn], axis=0)
    tt = lax.dot_general(ce5, m5, (((0,), (0,)), ((), ())),
                         preferred_element_type=jnp.float32)
    tt = jnp.maximum(tt, 1e-9)
    hinge = jnp.maximum(1.0 - tt * lax.rsqrt(tt), 0.0)
    part = jnp.sum(hinge, axis=0)[None, None]

    @pl.when(t == 0)
    def _():
        out_ref[...] = part

    @pl.when(t != 0)
    def _():
        out_ref[...] = out_ref[...] + part


def kernel(x, predCCoords, truthHitAssignementIdx, row_splits):
    del x, row_splits
    coords = predCCoords.astype(jnp.float32)
    tidx = truthHitAssignementIdx.reshape(-1).astype(jnp.int32)
    n = coords.shape[0]
    n_ev = n // 2
    c_q = float(np.arctanh(0.5) ** 2 + Q_MIN)

    chunk = -(-n_ev // (N_SUBCORES * N_LANES)) * N_LANES
    nep = chunk * N_SUBCORES
    pad = nep - n_ev
    t0 = jnp.concatenate([tidx[:n_ev], jnp.full((pad,), K_MAX, jnp.int32)])
    t1 = jnp.concatenate([tidx[n_ev:], jnp.full((pad,), K_MAX, jnp.int32)])
    tidx_flat = jnp.concatenate([t0, t1])
    cpad2 = jnp.pad(coords.T.reshape(3, 2, n_ev),
                    ((0, 0), (0, 0), (0, pad))).transpose(1, 0, 2)
    coords_flat = cpad2.reshape(-1)

    sc_out = _sc_stats_attrep(nep, chunk, c_q)(
        tidx_flat, coords_flat).reshape(2, 6, KPAD)
    stats = sc_out[:, :4, :]

    tiles_per_event = 4
    tile_n = nep // tiles_per_event
    assert tile_n * tiles_per_event == nep and tile_n % 128 == 0
    grid = 2 * tiles_per_event

    repall = pl.pallas_call(
        functools.partial(_tc_dense, tiles_per_event=tiles_per_event,
                          c_q=c_q),
        grid=(grid,),
        in_specs=[
            pl.BlockSpec((1, 3, tile_n),
                         lambda i: (i // (grid // 2), 0, i % (grid // 2))),
            pl.BlockSpec((1, 4, KPAD),
                         lambda i: (i // (grid // 2), 0, 0)),
        ],
        out_specs=pl.BlockSpec((1, 1, K_MAX),
                               lambda i: (i // (grid // 2), 0, 0)),
        out_shape=jax.ShapeDtypeStruct((2, 1, K_MAX), jnp.float32),
    )(cpad2, stats)[:, 0, :]

    nk = stats[:, 0, :K_MAX]
    att_s = sc_out[:, 4, :K_MAX]
    rep_o = sc_out[:, 5, :K_MAX]
    exists = (nk > 0).astype(jnp.float32)
    c2 = jnp.float32(c_q * c_q)
    inv = c_q / jnp.maximum(nk * c_q, 1e-6)
    mn = jnp.sum((stats[:, 1:4, :K_MAX] * inv[:, None, :]) ** 2, axis=1)
    repall = repall - float(pad) * jnp.maximum(
        1.0 - jnp.sqrt(mn + 1e-9), 0.0)
    att = c2 * att_s / jnp.maximum(nk, 1.0)
    rep = c2 * (repall - rep_o) / jnp.maximum(float(n_ev) - nk, 1.0)
    n_obj = jnp.maximum(jnp.sum(exists, axis=1), 1.0)
    v_att = jnp.sum(att * exists, axis=1) / n_obj
    v_rep = jnp.sum(rep * exists, axis=1) / n_obj
    return jnp.sum(v_att + v_rep) / 2.0


# speedup vs baseline: 1.9239x; 1.1221x over previous
import functools

import jax
import jax.numpy as jnp
import numpy as np
from jax import lax
from jax.experimental import pallas as pl
from jax.experimental.pallas import tpu as pltpu
from jax.experimental.pallas import tpu_sc as plsc

Q_MIN = 1.0
K_MAX = 256
KPAD = 272
N_LANES = 16
N_SUBCORES = 16
N_CORES = 2


def _sc_stats(nep, chunk):
    groups = chunk // N_LANES
    mesh = plsc.VectorSubcoreMesh(core_axis_name="c", subcore_axis_name="s")

    @functools.partial(
        pl.kernel,
        mesh=mesh,
        out_type=jax.ShapeDtypeStruct((N_CORES * 4 * KPAD,), jnp.float32),
        compiler_params=pltpu.CompilerParams(needs_layout_passes=False),
        scratch_types=[
            pltpu.VMEM((chunk,), jnp.int32),
            pltpu.VMEM((3 * chunk,), jnp.float32),
            pltpu.VMEM((N_LANES * KPAD,), jnp.float32),
            pltpu.VMEM((N_LANES * KPAD,), jnp.float32),
            pltpu.VMEM((N_LANES * KPAD,), jnp.float32),
            pltpu.VMEM((N_LANES * KPAD,), jnp.float32),
            pltpu.VMEM((4 * KPAD,), jnp.float32),
            pltpu.VMEM_SHARED((N_SUBCORES * 4 * KPAD,), jnp.float32),
            pltpu.VMEM((N_SUBCORES * 4 * KPAD,), jnp.float32),
        ],
    )
    def sc_kernel(tidx_hbm, coords_hbm, out_hbm,
                  idx_v, crd_v, acc0, acc1, acc2, acc3, red4_v, shared,
                  gath_v):
        c = lax.axis_index("c")
        s = lax.axis_index("s")

        base = c * nep + s * chunk
        pltpu.sync_copy(tidx_hbm.at[pl.ds(base, chunk)], idx_v)
        for d in range(3):
            pltpu.sync_copy(
                coords_hbm.at[pl.ds((c * 3 + d) * nep + s * chunk, chunk)],
                crd_v.at[pl.ds(d * chunk, chunk)])

        zeros16 = jnp.zeros((N_LANES,), jnp.float32)
        lane_off = lax.iota(jnp.int32, N_LANES) * KPAD
        ones16 = jnp.ones((N_LANES,), jnp.float32)

        def zero4_body(j, carry):
            sl = pl.ds(j * N_LANES, N_LANES)
            acc0[sl] = zeros16
            acc1[sl] = zeros16
            acc2[sl] = zeros16
            acc3[sl] = zeros16
            return carry

        lax.fori_loop(0, KPAD, zero4_body, 0)

        def scat_body(g, carry):
            sl = pl.ds(g * N_LANES, N_LANES)
            fidx = idx_v[sl] + lane_off
            plsc.addupdate_scatter(acc0, [fidx], ones16)
            plsc.addupdate_scatter(acc1, [fidx], crd_v[pl.ds(g * N_LANES, N_LANES)])
            plsc.addupdate_scatter(acc2, [fidx], crd_v[pl.ds(chunk + g * N_LANES, N_LANES)])
            plsc.addupdate_scatter(acc3, [fidx], crd_v[pl.ds(2 * chunk + g * N_LANES, N_LANES)])
            return carry

        lax.fori_loop(0, groups, scat_body, 0)

        def lred4_body(j, carry):
            for q, acc in enumerate((acc0, acc1, acc2, acc3)):
                v = acc[pl.ds(j * N_LANES, N_LANES)]
                for l in range(1, N_LANES):
                    v = v + acc[pl.ds(l * KPAD + j * N_LANES, N_LANES)]
                red4_v[pl.ds(q * KPAD + j * N_LANES, N_LANES)] = v
            return carry

        lax.fori_loop(0, KPAD // N_LANES, lred4_body, 0)

        pltpu.sync_copy(red4_v, shared.at[pl.ds(s * 4 * KPAD, 4 * KPAD)])
        plsc.subcore_barrier()

        @pl.when(s == 0)
        def _():
            pltpu.sync_copy(shared, gath_v)

            def tred4_body(j, carry):
                for q in range(4):
                    off = q * KPAD + j * N_LANES
                    v = gath_v[pl.ds(off, N_LANES)]
                    for t in range(1, N_SUBCORES):
                        v = v + gath_v[pl.ds(t * 4 * KPAD + off, N_LANES)]
                    red4_v[pl.ds(off, N_LANES)] = v
                return carry

            lax.fori_loop(0, KPAD // N_LANES, tred4_body, 0)
            pltpu.sync_copy(red4_v, out_hbm.at[pl.ds(c * 4 * KPAD, 4 * KPAD)])

    return sc_kernel


def _sc_attrep(nep, chunk, c_q):
    groups = chunk // N_LANES
    mesh = plsc.VectorSubcoreMesh(core_axis_name="c", subcore_axis_name="s")

    @functools.partial(
        pl.kernel,
        mesh=mesh,
        out_type=jax.ShapeDtypeStruct((N_CORES * 2 * KPAD,), jnp.float32),
        compiler_params=pltpu.CompilerParams(needs_layout_passes=False),
        scratch_types=[
            pltpu.VMEM((chunk,), jnp.int32),
            pltpu.VMEM((3 * chunk,), jnp.float32),
            pltpu.VMEM((N_LANES * KPAD,), jnp.float32),
            pltpu.VMEM((N_LANES * KPAD,), jnp.float32),
            pltpu.VMEM((4 * KPAD,), jnp.float32),
            pltpu.VMEM((2 * KPAD,), jnp.float32),
            pltpu.VMEM((3 * KPAD,), jnp.float32),
            pltpu.VMEM_SHARED((N_SUBCORES * 2 * KPAD,), jnp.float32),
            pltpu.VMEM((N_SUBCORES * 2 * KPAD,), jnp.float32),
        ],
    )
    def sc_kernel(tidx_hbm, coords_hbm, stats_hbm, out_hbm,
                  idx_v, crd_v, acc0, acc1, stats_v, red2_v, ctr_v, shared,
                  gath_v):
        c = lax.axis_index("c")
        s = lax.axis_index("s")

        base = c * nep + s * chunk
        pltpu.sync_copy(tidx_hbm.at[pl.ds(base, chunk)], idx_v)
        for d in range(3):
            pltpu.sync_copy(
                coords_hbm.at[pl.ds((c * 3 + d) * nep + s * chunk, chunk)],
                crd_v.at[pl.ds(d * chunk, chunk)])
        pltpu.sync_copy(stats_hbm.at[pl.ds(c * 4 * KPAD, 4 * KPAD)], stats_v)

        zeros16 = jnp.zeros((N_LANES,), jnp.float32)
        lane_off = lax.iota(jnp.int32, N_LANES) * KPAD

        def ctr_body(j, carry):
            sl = pl.ds(j * N_LANES, N_LANES)
            nk = stats_v[sl]
            inv = c_q / jnp.maximum(nk * c_q, 1e-6)
            for d in range(3):
                ctr_v[pl.ds(d * KPAD + j * N_LANES, N_LANES)] = (
                    stats_v[pl.ds((1 + d) * KPAD + j * N_LANES, N_LANES)] * inv)
            return carry

        lax.fori_loop(0, KPAD // N_LANES, ctr_body, 0)

        def zero2_body(j, carry):
            sl = pl.ds(j * N_LANES, N_LANES)
            acc0[sl] = zeros16
            acc1[sl] = zeros16
            return carry

        lax.fori_loop(0, KPAD, zero2_body, 0)

        magic = jnp.full((N_LANES,), 0x5F3759DF, jnp.int32)

        def hit_body(g, carry):
            sl = pl.ds(g * N_LANES, N_LANES)
            ti = idx_v[sl]
            dx = crd_v[pl.ds(g * N_LANES, N_LANES)] - plsc.load_gather(ctr_v, [ti])
            dy = crd_v[pl.ds(chunk + g * N_LANES, N_LANES)] - plsc.load_gather(
                ctr_v, [ti + KPAD])
            dz = crd_v[pl.ds(2 * chunk + g * N_LANES, N_LANES)] - plsc.load_gather(
                ctr_v, [ti + 2 * KPAD])
            d2 = dx * dx + dy * dy + dz * dz
            fidx = ti + lane_off
            plsc.addupdate_scatter(acc0, [fidx], d2)
            t = d2 + 1e-9
            th = t * 0.5
            y = plsc.bitcast(magic - (plsc.bitcast(t, jnp.int32) >> 1),
                             jnp.float32)
            y = y * (1.5 - th * y * y)
            y = y * (1.5 - th * y * y)
            hinge = jnp.maximum(1.0 - t * y, 0.0)
            plsc.addupdate_scatter(acc1, [fidx], hinge)
            return carry

        lax.fori_loop(0, groups, hit_body, 0)

        def lred2_body(j, carry):
            for q, acc in enumerate((acc0, acc1)):
                v = acc[pl.ds(j * N_LANES, N_LANES)]
                for l in range(1, N_LANES):
                    v = v + acc[pl.ds(l * KPAD + j * N_LANES, N_LANES)]
                red2_v[pl.ds(q * KPAD + j * N_LANES, N_LANES)] = v
            return carry

        lax.fori_loop(0, KPAD // N_LANES, lred2_body, 0)

        pltpu.sync_copy(red2_v, shared.at[pl.ds(s * 2 * KPAD, 2 * KPAD)])
        plsc.subcore_barrier()

        @pl.when(s == 0)
        def _():
            pltpu.sync_copy(shared, gath_v)

            def tred2_body(j, carry):
                for q in range(2):
                    off = q * KPAD + j * N_LANES
                    v = gath_v[pl.ds(off, N_LANES)]
                    for t in range(1, N_SUBCORES):
                        v = v + gath_v[pl.ds(t * 2 * KPAD + off, N_LANES)]
                    red2_v[pl.ds(off, N_LANES)] = v
                return carry

            lax.fori_loop(0, KPAD // N_LANES, tred2_body, 0)
            pltpu.sync_copy(red2_v, out_hbm.at[pl.ds(c * 2 * KPAD, 2 * KPAD)])

    return sc_kernel


def _tc_dense(coords_ref, stats_ref, out_ref, *, tiles_per_event, c_q):
    i = pl.program_id(0)
    t = lax.rem(i, tiles_per_event)

    nk = stats_ref[0, 0, :K_MAX]
    inv = c_q / jnp.maximum(nk * c_q, 1e-6)
    mx = stats_ref[0, 1, :K_MAX] * inv
    my = stats_ref[0, 2, :K_MAX] * inv
    mz = stats_ref[0, 3, :K_MAX] * inv
    mn = mx * mx + my * my + mz * mz
    m5 = jnp.stack([-2.0 * mx, -2.0 * my, -2.0 * mz, mn,
                    jnp.ones_like(mn)])
    ce = coords_ref[0]
    cn = jnp.sum(ce * ce, axis=0, keepdims=True) + 1e-9
    ce5 = jnp.concatenate([ce, jnp.ones_like(cn), cn], axis=0)
    tt = lax.dot_general(ce5, m5, (((0,), (0,)), ((), ())),
                         preferred_element_type=jnp.float32)
    tt = jnp.maximum(tt, 1e-9)
    hinge = jnp.maximum(1.0 - tt * lax.rsqrt(tt), 0.0)
    part = jnp.sum(hinge, axis=0)[None, None]

    @pl.when(t == 0)
    def _():
        out_ref[...] = part

    @pl.when(t != 0)
    def _():
        out_ref[...] = out_ref[...] + part


def kernel(x, predCCoords, truthHitAssignementIdx, row_splits):
    del x, row_splits
    coords = predCCoords.astype(jnp.float32)
    tidx = truthHitAssignementIdx.reshape(-1).astype(jnp.int32)
    n = coords.shape[0]
    n_ev = n // 2
    c_q = float(np.arctanh(0.5) ** 2 + Q_MIN)

    chunk = -(-n_ev // (N_SUBCORES * N_LANES)) * N_LANES
    nep = chunk * N_SUBCORES
    pad = nep - n_ev
    t0 = jnp.concatenate([tidx[:n_ev], jnp.full((pad,), K_MAX, jnp.int32)])
    t1 = jnp.concatenate([tidx[n_ev:], jnp.full((pad,), K_MAX, jnp.int32)])
    tidx_flat = jnp.concatenate([t0, t1])
    cpad2 = jnp.pad(coords.T.reshape(3, 2, n_ev),
                    ((0, 0), (0, 0), (0, pad))).transpose(1, 0, 2)
    coords_flat = cpad2.reshape(-1)

    stats_flat = _sc_stats(nep, chunk)(tidx_flat, coords_flat)
    stats = stats_flat.reshape(2, 4, KPAD)
    attrep = _sc_attrep(nep, chunk, c_q)(
        tidx_flat, coords_flat, stats_flat).reshape(2, 2, KPAD)

    tiles_per_event = 4
    tile_n = nep // tiles_per_event
    assert tile_n * tiles_per_event == nep and tile_n % 128 == 0
    grid = 2 * tiles_per_event

    repall = pl.pallas_call(
        functools.partial(_tc_dense, tiles_per_event=tiles_per_event,
                          c_q=c_q),
        grid=(grid,),
        in_specs=[
            pl.BlockSpec((1, 3, tile_n),
                         lambda i: (i // (grid // 2), 0, i % (grid // 2))),
            pl.BlockSpec((1, 4, KPAD),
                         lambda i: (i // (grid // 2), 0, 0)),
        ],
        out_specs=pl.BlockSpec((1, 1, K_MAX),
                               lambda i: (i // (grid // 2), 0, 0)),
        out_shape=jax.ShapeDtypeStruct((2, 1, K_MAX), jnp.float32),
    )(cpad2, stats)[:, 0, :]

    nk = stats[:, 0, :K_MAX]
    att_s = attrep[:, 0, :K_MAX]
    rep_o = attrep[:, 1, :K_MAX]
    exists = (nk > 0).astype(jnp.float32)
    c2 = jnp.float32(c_q * c_q)
    inv = c_q / jnp.maximum(nk * c_q, 1e-6)
    mn = jnp.sum((stats[:, 1:4, :K_MAX] * inv[:, None, :]) ** 2, axis=1)
    repall = repall - float(pad) * jnp.maximum(
        1.0 - jnp.sqrt(mn + 1e-9), 0.0)
    att = c2 * att_s / jnp.maximum(nk, 1.0)
    rep = c2 * (repall - rep_o) / jnp.maximum(float(n_ev) - nk, 1.0)
    n_obj = jnp.maximum(jnp.sum(exists, axis=1), 1.0)
    v_att = jnp.sum(att * exists, axis=1) / n_obj
    v_rep = jnp.sum(rep * exists, axis=1) / n_obj
    return jnp.sum(v_att + v_rep) / 2.0


# speedup vs baseline: 1.9246x; 1.0003x over previous
import functools

import jax
import jax.numpy as jnp
import numpy as np
from jax import lax
from jax.experimental import pallas as pl
from jax.experimental.pallas import tpu as pltpu
from jax.experimental.pallas import tpu_sc as plsc

Q_MIN = 1.0
K_MAX = 256
KPAD = 272
N_LANES = 16
N_SUBCORES = 16
N_CORES = 2


def _sc_stats(nep, chunk):
    groups = chunk // N_LANES
    mesh = plsc.VectorSubcoreMesh(core_axis_name="c", subcore_axis_name="s")

    @functools.partial(
        pl.kernel,
        mesh=mesh,
        out_type=jax.ShapeDtypeStruct((N_CORES * 4 * KPAD,), jnp.float32),
        compiler_params=pltpu.CompilerParams(needs_layout_passes=False),
        scratch_types=[
            pltpu.VMEM((chunk,), jnp.int32),
            pltpu.VMEM((3 * chunk,), jnp.float32),
            pltpu.VMEM((N_LANES * KPAD,), jnp.float32),
            pltpu.VMEM((N_LANES * KPAD,), jnp.float32),
            pltpu.VMEM((N_LANES * KPAD,), jnp.float32),
            pltpu.VMEM((N_LANES * KPAD,), jnp.float32),
            pltpu.VMEM((4 * KPAD,), jnp.float32),
            pltpu.VMEM_SHARED((N_SUBCORES * 4 * KPAD,), jnp.float32),
            pltpu.VMEM((N_SUBCORES * 4 * KPAD,), jnp.float32),
        ],
    )
    def sc_kernel(tidx_hbm, coords_hbm, out_hbm,
                  idx_v, crd_v, acc0, acc1, acc2, acc3, red4_v, shared,
                  gath_v):
        c = lax.axis_index("c")
        s = lax.axis_index("s")

        base = c * nep + s * chunk
        pltpu.sync_copy(tidx_hbm.at[pl.ds(base, chunk)], idx_v)
        for d in range(3):
            pltpu.sync_copy(
                coords_hbm.at[pl.ds((c * 3 + d) * nep + s * chunk, chunk)],
                crd_v.at[pl.ds(d * chunk, chunk)])

        zeros16 = jnp.zeros((N_LANES,), jnp.float32)
        lane_off = lax.iota(jnp.int32, N_LANES) * KPAD
        ones16 = jnp.ones((N_LANES,), jnp.float32)

        def zero4_body(j, carry):
            sl = pl.ds(j * N_LANES, N_LANES)
            acc0[sl] = zeros16
            acc1[sl] = zeros16
            acc2[sl] = zeros16
            acc3[sl] = zeros16
            return carry

        lax.fori_loop(0, KPAD, zero4_body, 0)

        def scat_body(g2, carry):
            for g0 in range(2):
                g = g2 * 2 + g0
                sl = pl.ds(g * N_LANES, N_LANES)
                fidx = idx_v[sl] + lane_off
                plsc.addupdate_scatter(acc0, [fidx], ones16)
                plsc.addupdate_scatter(acc1, [fidx], crd_v[pl.ds(g * N_LANES, N_LANES)])
                plsc.addupdate_scatter(acc2, [fidx], crd_v[pl.ds(chunk + g * N_LANES, N_LANES)])
                plsc.addupdate_scatter(acc3, [fidx], crd_v[pl.ds(2 * chunk + g * N_LANES, N_LANES)])
            return carry

        assert groups % 2 == 0
        lax.fori_loop(0, groups // 2, scat_body, 0)

        def lred4_body(j, carry):
            for q, acc in enumerate((acc0, acc1, acc2, acc3)):
                v = acc[pl.ds(j * N_LANES, N_LANES)]
                for l in range(1, N_LANES):
                    v = v + acc[pl.ds(l * KPAD + j * N_LANES, N_LANES)]
                red4_v[pl.ds(q * KPAD + j * N_LANES, N_LANES)] = v
            return carry

        lax.fori_loop(0, KPAD // N_LANES, lred4_body, 0)

        pltpu.sync_copy(red4_v, shared.at[pl.ds(s * 4 * KPAD, 4 * KPAD)])
        plsc.subcore_barrier()

        @pl.when(s == 0)
        def _():
            pltpu.sync_copy(shared, gath_v)

            def tred4_body(j, carry):
                for q in range(4):
                    off = q * KPAD + j * N_LANES
                    v = gath_v[pl.ds(off, N_LANES)]
                    for t in range(1, N_SUBCORES):
                        v = v + gath_v[pl.ds(t * 4 * KPAD + off, N_LANES)]
                    red4_v[pl.ds(off, N_LANES)] = v
                return carry

            lax.fori_loop(0, KPAD // N_LANES, tred4_body, 0)
            pltpu.sync_copy(red4_v, out_hbm.at[pl.ds(c * 4 * KPAD, 4 * KPAD)])

    return sc_kernel


def _sc_attrep(nep, chunk, c_q):
    groups = chunk // N_LANES
    mesh = plsc.VectorSubcoreMesh(core_axis_name="c", subcore_axis_name="s")

    @functools.partial(
        pl.kernel,
        mesh=mesh,
        out_type=jax.ShapeDtypeStruct((N_CORES * 2 * KPAD,), jnp.float32),
        compiler_params=pltpu.CompilerParams(needs_layout_passes=False),
        scratch_types=[
            pltpu.VMEM((chunk,), jnp.int32),
            pltpu.VMEM((3 * chunk,), jnp.float32),
            pltpu.VMEM((N_LANES * KPAD,), jnp.float32),
            pltpu.VMEM((N_LANES * KPAD,), jnp.float32),
            pltpu.VMEM((4 * KPAD,), jnp.float32),
            pltpu.VMEM((2 * KPAD,), jnp.float32),
            pltpu.VMEM((3 * KPAD,), jnp.float32),
            pltpu.VMEM_SHARED((N_SUBCORES * 2 * KPAD,), jnp.float32),
            pltpu.VMEM((N_SUBCORES * 2 * KPAD,), jnp.float32),
        ],
    )
    def sc_kernel(tidx_hbm, coords_hbm, stats_hbm, out_hbm,
                  idx_v, crd_v, acc0, acc1, stats_v, red2_v, ctr_v, shared,
                  gath_v):
        c = lax.axis_index("c")
        s = lax.axis_index("s")

        base = c * nep + s * chunk
        pltpu.sync_copy(tidx_hbm.at[pl.ds(base, chunk)], idx_v)
        for d in range(3):
            pltpu.sync_copy(
                coords_hbm.at[pl.ds((c * 3 + d) * nep + s * chunk, chunk)],
                crd_v.at[pl.ds(d * chunk, chunk)])
        pltpu.sync_copy(stats_hbm.at[pl.ds(c * 4 * KPAD, 4 * KPAD)], stats_v)

        zeros16 = jnp.zeros((N_LANES,), jnp.float32)
        lane_off = lax.iota(jnp.int32, N_LANES) * KPAD

        def ctr_body(j, carry):
            sl = pl.ds(j * N_LANES, N_LANES)
            nk = stats_v[sl]
            inv = c_q / jnp.maximum(nk * c_q, 1e-6)
            for d in range(3):
                ctr_v[pl.ds(d * KPAD + j * N_LANES, N_LANES)] = (
                    stats_v[pl.ds((1 + d) * KPAD + j * N_LANES, N_LANES)] * inv)
            return carry

        lax.fori_loop(0, KPAD // N_LANES, ctr_body, 0)

        def zero2_body(j, carry):
            sl = pl.ds(j * N_LANES, N_LANES)
            acc0[sl] = zeros16
            acc1[sl] = zeros16
            return carry

        lax.fori_loop(0, KPAD, zero2_body, 0)

        magic = jnp.full((N_LANES,), 0x5F3759DF, jnp.int32)

        def hit_body(g2, carry):
          for g0 in range(2):
            g = g2 * 2 + g0
            sl = pl.ds(g * N_LANES, N_LANES)
            ti = idx_v[sl]
            dx = crd_v[pl.ds(g * N_LANES, N_LANES)] - plsc.load_gather(ctr_v, [ti])
            dy = crd_v[pl.ds(chunk + g * N_LANES, N_LANES)] - plsc.load_gather(
                ctr_v, [ti + KPAD])
            dz = crd_v[pl.ds(2 * chunk + g * N_LANES, N_LANES)] - plsc.load_gather(
                ctr_v, [ti + 2 * KPAD])
            d2 = dx * dx + dy * dy + dz * dz
            fidx = ti + lane_off
            plsc.addupdate_scatter(acc0, [fidx], d2)
            t = d2 + 1e-9
            th = t * 0.5
            y = plsc.bitcast(magic - (plsc.bitcast(t, jnp.int32) >> 1),
                             jnp.float32)
            y = y * (1.5 - th * y * y)
            y = y * (1.5 - th * y * y)
            hinge = jnp.maximum(1.0 - t * y, 0.0)
            plsc.addupdate_scatter(acc1, [fidx], hinge)
          return carry

        lax.fori_loop(0, groups // 2, hit_body, 0)

        def lred2_body(j, carry):
            for q, acc in enumerate((acc0, acc1)):
                v = acc[pl.ds(j * N_LANES, N_LANES)]
                for l in range(1, N_LANES):
                    v = v + acc[pl.ds(l * KPAD + j * N_LANES, N_LANES)]
                red2_v[pl.ds(q * KPAD + j * N_LANES, N_LANES)] = v
            return carry

        lax.fori_loop(0, KPAD // N_LANES, lred2_body, 0)

        pltpu.sync_copy(red2_v, shared.at[pl.ds(s * 2 * KPAD, 2 * KPAD)])
        plsc.subcore_barrier()

        @pl.when(s == 0)
        def _():
            pltpu.sync_copy(shared, gath_v)

            def tred2_body(j, carry):
                for q in range(2):
                    off = q * KPAD + j * N_LANES
                    v = gath_v[pl.ds(off, N_LANES)]
                    for t in range(1, N_SUBCORES):
                        v = v + gath_v[pl.ds(t * 2 * KPAD + off, N_LANES)]
                    red2_v[pl.ds(off, N_LANES)] = v
                return carry

            lax.fori_loop(0, KPAD // N_LANES, tred2_body, 0)
            pltpu.sync_copy(red2_v, out_hbm.at[pl.ds(c * 2 * KPAD, 2 * KPAD)])

    return sc_kernel


def _tc_dense(coords_ref, stats_ref, out_ref, *, tiles_per_event, c_q):
    i = pl.program_id(0)
    t = lax.rem(i, tiles_per_event)

    nk = stats_ref[0, 0, :K_MAX]
    inv = c_q / jnp.maximum(nk * c_q, 1e-6)
    mx = stats_ref[0, 1, :K_MAX] * inv
    my = stats_ref[0, 2, :K_MAX] * inv
    mz = stats_ref[0, 3, :K_MAX] * inv
    mn = mx * mx + my * my + mz * mz
    m5 = jnp.stack([-2.0 * mx, -2.0 * my, -2.0 * mz, mn,
                    jnp.ones_like(mn)])
    ce = coords_ref[0]
    cn = jnp.sum(ce * ce, axis=0, keepdims=True) + 1e-9
    ce5 = jnp.concatenate([ce, jnp.ones_like(cn), cn], axis=0)
    tt = lax.dot_general(ce5, m5, (((0,), (0,)), ((), ())),
                         preferred_element_type=jnp.float32)
    tt = jnp.maximum(tt, 1e-9)
    hinge = jnp.maximum(1.0 - tt * lax.rsqrt(tt), 0.0)
    part = jnp.sum(hinge, axis=0)[None, None]

    @pl.when(t == 0)
    def _():
        out_ref[...] = part

    @pl.when(t != 0)
    def _():
        out_ref[...] = out_ref[...] + part


def kernel(x, predCCoords, truthHitAssignementIdx, row_splits):
    del x, row_splits
    coords = predCCoords.astype(jnp.float32)
    tidx = truthHitAssignementIdx.reshape(-1).astype(jnp.int32)
    n = coords.shape[0]
    n_ev = n // 2
    c_q = float(np.arctanh(0.5) ** 2 + Q_MIN)

    chunk = -(-n_ev // (N_SUBCORES * N_LANES)) * N_LANES
    nep = chunk * N_SUBCORES
    pad = nep - n_ev
    t0 = jnp.concatenate([tidx[:n_ev], jnp.full((pad,), K_MAX, jnp.int32)])
    t1 = jnp.concatenate([tidx[n_ev:], jnp.full((pad,), K_MAX, jnp.int32)])
    tidx_flat = jnp.concatenate([t0, t1])
    cpad2 = jnp.pad(coords.T.reshape(3, 2, n_ev),
                    ((0, 0), (0, 0), (0, pad))).transpose(1, 0, 2)
    coords_flat = cpad2.reshape(-1)

    stats_flat = _sc_stats(nep, chunk)(tidx_flat, coords_flat)
    stats = stats_flat.reshape(2, 4, KPAD)
    attrep = _sc_attrep(nep, chunk, c_q)(
        tidx_flat, coords_flat, stats_flat).reshape(2, 2, KPAD)

    tiles_per_event = 4
    tile_n = nep // tiles_per_event
    assert tile_n * tiles_per_event == nep and tile_n % 128 == 0
    grid = 2 * tiles_per_event

    repall = pl.pallas_call(
        functools.partial(_tc_dense, tiles_per_event=tiles_per_event,
                          c_q=c_q),
        grid=(grid,),
        in_specs=[
            pl.BlockSpec((1, 3, tile_n),
                         lambda i: (i // (grid // 2), 0, i % (grid // 2))),
            pl.BlockSpec((1, 4, KPAD),
                         lambda i: (i // (grid // 2), 0, 0)),
        ],
        out_specs=pl.BlockSpec((1, 1, K_MAX),
                               lambda i: (i // (grid // 2), 0, 0)),
        out_shape=jax.ShapeDtypeStruct((2, 1, K_MAX), jnp.float32),
    )(cpad2, stats)[:, 0, :]

    nk = stats[:, 0, :K_MAX]
    att_s = attrep[:, 0, :K_MAX]
    rep_o = attrep[:, 1, :K_MAX]
    exists = (nk > 0).astype(jnp.float32)
    c2 = jnp.float32(c_q * c_q)
    inv = c_q / jnp.maximum(nk * c_q, 1e-6)
    mn = jnp.sum((stats[:, 1:4, :K_MAX] * inv[:, None, :]) ** 2, axis=1)
    repall = repall - float(pad) * jnp.maximum(
        1.0 - jnp.sqrt(mn + 1e-9), 0.0)
    att = c2 * att_s / jnp.maximum(nk, 1.0)
    rep = c2 * (repall - rep_o) / jnp.maximum(float(n_ev) - nk, 1.0)
    n_obj = jnp.maximum(jnp.sum(exists, axis=1), 1.0)
    v_att = jnp.sum(att * exists, axis=1) / n_obj
    v_rep = jnp.sum(rep * exists, axis=1) / n_obj
    return jnp.sum(v_att + v_rep) / 2.0


# speedup vs baseline: 1.9280x; 1.0018x over previous
import functools

import jax
import jax.numpy as jnp
import numpy as np
from jax import lax
from jax.experimental import pallas as pl
from jax.experimental.pallas import tpu as pltpu
from jax.experimental.pallas import tpu_sc as plsc

Q_MIN = 1.0
K_MAX = 256
KPAD = 272
N_LANES = 16
N_SUBCORES = 16
N_CORES = 2


def _sc_stats(nep, chunk):
    groups = chunk // N_LANES
    mesh = plsc.VectorSubcoreMesh(core_axis_name="c", subcore_axis_name="s")

    @functools.partial(
        pl.kernel,
        mesh=mesh,
        out_type=jax.ShapeDtypeStruct((N_CORES * 4 * KPAD,), jnp.float32),
        compiler_params=pltpu.CompilerParams(needs_layout_passes=False),
        scratch_types=[
            pltpu.VMEM((chunk,), jnp.int32),
            pltpu.VMEM((3 * chunk,), jnp.float32),
            pltpu.VMEM((N_LANES * KPAD,), jnp.float32),
            pltpu.VMEM((N_LANES * KPAD,), jnp.float32),
            pltpu.VMEM((N_LANES * KPAD,), jnp.float32),
            pltpu.VMEM((N_LANES * KPAD,), jnp.float32),
            pltpu.VMEM((4 * KPAD,), jnp.float32),
            pltpu.VMEM_SHARED((N_SUBCORES * 4 * KPAD,), jnp.float32),
            pltpu.VMEM((N_SUBCORES * 4 * KPAD,), jnp.float32),
        ],
    )
    def sc_kernel(tidx_hbm, coords_hbm, out_hbm,
                  idx_v, crd_v, acc0, acc1, acc2, acc3, red4_v, shared,
                  gath_v):
        c = lax.axis_index("c")
        s = lax.axis_index("s")

        base = c * nep + s * chunk
        pltpu.sync_copy(tidx_hbm.at[pl.ds(base, chunk)], idx_v)
        for d in range(3):
            pltpu.sync_copy(
                coords_hbm.at[pl.ds((c * 3 + d) * nep + s * chunk, chunk)],
                crd_v.at[pl.ds(d * chunk, chunk)])

        zeros16 = jnp.zeros((N_LANES,), jnp.float32)
        lane_off = lax.iota(jnp.int32, N_LANES) * KPAD
        ones16 = jnp.ones((N_LANES,), jnp.float32)

        def zero4_body(j, carry):
            sl = pl.ds(j * N_LANES, N_LANES)
            acc0[sl] = zeros16
            acc1[sl] = zeros16
            acc2[sl] = zeros16
            acc3[sl] = zeros16
            return carry

        lax.fori_loop(0, KPAD, zero4_body, 0)

        def scat_body(g2, carry):
            for g0 in range(2):
                g = g2 * 2 + g0
                sl = pl.ds(g * N_LANES, N_LANES)
                fidx = idx_v[sl] + lane_off
                plsc.addupdate_scatter(acc0, [fidx], ones16)
                plsc.addupdate_scatter(acc1, [fidx], crd_v[pl.ds(g * N_LANES, N_LANES)])
                plsc.addupdate_scatter(acc2, [fidx], crd_v[pl.ds(chunk + g * N_LANES, N_LANES)])
                plsc.addupdate_scatter(acc3, [fidx], crd_v[pl.ds(2 * chunk + g * N_LANES, N_LANES)])
            return carry

        assert groups % 2 == 0
        lax.fori_loop(0, groups // 2, scat_body, 0)

        def lred4_body(j, carry):
            for q, acc in enumerate((acc0, acc1, acc2, acc3)):
                v = acc[pl.ds(j * N_LANES, N_LANES)]
                for l in range(1, N_LANES):
                    v = v + acc[pl.ds(l * KPAD + j * N_LANES, N_LANES)]
                red4_v[pl.ds(q * KPAD + j * N_LANES, N_LANES)] = v
            return carry

        lax.fori_loop(0, KPAD // N_LANES, lred4_body, 0)

        pltpu.sync_copy(red4_v, shared.at[pl.ds(s * 4 * KPAD, 4 * KPAD)])
        plsc.subcore_barrier()

        @pl.when(s == 0)
        def _():
            pltpu.sync_copy(shared, gath_v)

            def tred4_body(j, carry):
                for q in range(4):
                    off = q * KPAD + j * N_LANES
                    v = gath_v[pl.ds(off, N_LANES)]
                    for t in range(1, N_SUBCORES):
                        v = v + gath_v[pl.ds(t * 4 * KPAD + off, N_LANES)]
                    red4_v[pl.ds(off, N_LANES)] = v
                return carry

            lax.fori_loop(0, KPAD // N_LANES, tred4_body, 0)
            pltpu.sync_copy(red4_v, out_hbm.at[pl.ds(c * 4 * KPAD, 4 * KPAD)])

    return sc_kernel


def _sc_attrep(nep, chunk, c_q):
    groups = chunk // N_LANES
    mesh = plsc.VectorSubcoreMesh(core_axis_name="c", subcore_axis_name="s")

    @functools.partial(
        pl.kernel,
        mesh=mesh,
        out_type=jax.ShapeDtypeStruct((N_CORES * 2 * KPAD,), jnp.float32),
        compiler_params=pltpu.CompilerParams(needs_layout_passes=False),
        scratch_types=[
            pltpu.VMEM((chunk,), jnp.int32),
            pltpu.VMEM((3 * chunk,), jnp.float32),
            pltpu.VMEM((N_LANES * KPAD,), jnp.float32),
            pltpu.VMEM((N_LANES * KPAD,), jnp.float32),
            pltpu.VMEM((4 * KPAD,), jnp.float32),
            pltpu.VMEM((2 * KPAD,), jnp.float32),
            pltpu.VMEM((3 * KPAD,), jnp.float32),
            pltpu.VMEM_SHARED((N_SUBCORES * 2 * KPAD,), jnp.float32),
            pltpu.VMEM((N_SUBCORES * 2 * KPAD,), jnp.float32),
        ],
    )
    def sc_kernel(tidx_hbm, coords_hbm, stats_hbm, out_hbm,
                  idx_v, crd_v, acc0, acc1, stats_v, red2_v, ctr_v, shared,
                  gath_v):
        c = lax.axis_index("c")
        s = lax.axis_index("s")

        base = c * nep + s * chunk
        pltpu.sync_copy(tidx_hbm.at[pl.ds(base, chunk)], idx_v)
        for d in range(3):
            pltpu.sync_copy(
                coords_hbm.at[pl.ds((c * 3 + d) * nep + s * chunk, chunk)],
                crd_v.at[pl.ds(d * chunk, chunk)])
        pltpu.sync_copy(stats_hbm.at[pl.ds(c * 4 * KPAD, 4 * KPAD)], stats_v)

        zeros16 = jnp.zeros((N_LANES,), jnp.float32)
        lane_off = lax.iota(jnp.int32, N_LANES) * KPAD

        def ctr_body(j, carry):
            sl = pl.ds(j * N_LANES, N_LANES)
            nk = stats_v[sl]
            inv = c_q / jnp.maximum(nk * c_q, 1e-6)
            for d in range(3):
                ctr_v[pl.ds(d * KPAD + j * N_LANES, N_LANES)] = (
                    stats_v[pl.ds((1 + d) * KPAD + j * N_LANES, N_LANES)] * inv)
            return carry

        lax.fori_loop(0, KPAD // N_LANES, ctr_body, 0)

        def zero2_body(j, carry):
            sl = pl.ds(j * N_LANES, N_LANES)
            acc0[sl] = zeros16
            acc1[sl] = zeros16
            return carry

        lax.fori_loop(0, KPAD, zero2_body, 0)

        magic = jnp.full((N_LANES,), 0x5F3759DF, jnp.int32)

        def hit_body(g2, carry):
          for g0 in range(2):
            g = g2 * 2 + g0
            sl = pl.ds(g * N_LANES, N_LANES)
            ti = idx_v[sl]
            dx = crd_v[pl.ds(g * N_LANES, N_LANES)] - plsc.load_gather(ctr_v, [ti])
            dy = crd_v[pl.ds(chunk + g * N_LANES, N_LANES)] - plsc.load_gather(
                ctr_v, [ti + KPAD])
            dz = crd_v[pl.ds(2 * chunk + g * N_LANES, N_LANES)] - plsc.load_gather(
                ctr_v, [ti + 2 * KPAD])
            d2 = dx * dx + dy * dy + dz * dz
            fidx = ti + lane_off
            plsc.addupdate_scatter(acc0, [fidx], d2)
            t = d2 + 1e-9
            th = t * 0.5
            y = plsc.bitcast(magic - (plsc.bitcast(t, jnp.int32) >> 1),
                             jnp.float32)
            y = y * (1.5 - th * y * y)
            y = y * (1.5 - th * y * y)
            hinge = jnp.maximum(1.0 - t * y, 0.0)
            plsc.addupdate_scatter(acc1, [fidx], hinge)
          return carry

        lax.fori_loop(0, groups // 2, hit_body, 0)

        def lred2_body(j, carry):
            for q, acc in enumerate((acc0, acc1)):
                v = acc[pl.ds(j * N_LANES, N_LANES)]
                for l in range(1, N_LANES):
                    v = v + acc[pl.ds(l * KPAD + j * N_LANES, N_LANES)]
                red2_v[pl.ds(q * KPAD + j * N_LANES, N_LANES)] = v
            return carry

        lax.fori_loop(0, KPAD // N_LANES, lred2_body, 0)

        pltpu.sync_copy(red2_v, shared.at[pl.ds(s * 2 * KPAD, 2 * KPAD)])
        plsc.subcore_barrier()

        @pl.when(s == 0)
        def _():
            pltpu.sync_copy(shared, gath_v)

            def tred2_body(j, carry):
                for q in range(2):
                    off = q * KPAD + j * N_LANES
                    v = gath_v[pl.ds(off, N_LANES)]
                    for t in range(1, N_SUBCORES):
                        v = v + gath_v[pl.ds(t * 2 * KPAD + off, N_LANES)]
                    red2_v[pl.ds(off, N_LANES)] = v
                return carry

            lax.fori_loop(0, KPAD // N_LANES, tred2_body, 0)
            pltpu.sync_copy(red2_v, out_hbm.at[pl.ds(c * 2 * KPAD, 2 * KPAD)])

    return sc_kernel


def _tc_dense(coords_ref, stats_ref, out_ref, *, tiles_per_event, c_q):
    i = pl.program_id(0)
    t = lax.rem(i, tiles_per_event)

    nk = stats_ref[0, 0, :K_MAX]
    inv = c_q / jnp.maximum(nk * c_q, 1e-6)
    mx = stats_ref[0, 1, :K_MAX] * inv
    my = stats_ref[0, 2, :K_MAX] * inv
    mz = stats_ref[0, 3, :K_MAX] * inv
    mn = mx * mx + my * my + mz * mz
    m5 = jnp.stack([-2.0 * mx, -2.0 * my, -2.0 * mz, mn,
                    jnp.ones_like(mn)])
    ce = coords_ref[0]
    cn = jnp.sum(ce * ce, axis=0, keepdims=True) + 1e-9
    ce5 = jnp.concatenate([ce, jnp.ones_like(cn), cn], axis=0)
    tt = lax.dot_general(ce5, m5, (((0,), (0,)), ((), ())),
                         preferred_element_type=jnp.float32)
    tt = jnp.maximum(tt, 1e-9)
    hinge = jnp.maximum(1.0 - tt * lax.rsqrt(tt), 0.0)
    part = jnp.sum(hinge, axis=0)[None, None]

    @pl.when(t == 0)
    def _():
        out_ref[...] = part

    @pl.when(t != 0)
    def _():
        out_ref[...] = out_ref[...] + part


def kernel(x, predCCoords, truthHitAssignementIdx, row_splits):
    del x, row_splits
    coords = predCCoords.astype(jnp.float32)
    tidx = truthHitAssignementIdx.reshape(-1).astype(jnp.int32)
    n = coords.shape[0]
    n_ev = n // 2
    c_q = float(np.arctanh(0.5) ** 2 + Q_MIN)

    chunk = -(-n_ev // (N_SUBCORES * N_LANES)) * N_LANES
    nep = chunk * N_SUBCORES
    pad = nep - n_ev
    t0 = jnp.concatenate([tidx[:n_ev], jnp.full((pad,), K_MAX, jnp.int32)])
    t1 = jnp.concatenate([tidx[n_ev:], jnp.full((pad,), K_MAX, jnp.int32)])
    tidx_flat = jnp.concatenate([t0, t1])
    cpad2 = jnp.pad(coords.T.reshape(3, 2, n_ev),
                    ((0, 0), (0, 0), (0, pad))).transpose(1, 0, 2)
    coords_flat = cpad2.reshape(-1)

    stats_flat = _sc_stats(nep, chunk)(tidx_flat, coords_flat)
    stats = stats_flat.reshape(2, 4, KPAD)
    attrep = _sc_attrep(nep, chunk, c_q)(
        tidx_flat, coords_flat, stats_flat).reshape(2, 2, KPAD)

    tiles_per_event = 2
    tile_n = nep // tiles_per_event
    assert tile_n * tiles_per_event == nep and tile_n % 128 == 0
    grid = 2 * tiles_per_event

    repall = pl.pallas_call(
        functools.partial(_tc_dense, tiles_per_event=tiles_per_event,
                          c_q=c_q),
        grid=(grid,),
        in_specs=[
            pl.BlockSpec((1, 3, tile_n),
                         lambda i: (i // (grid // 2), 0, i % (grid // 2))),
            pl.BlockSpec((1, 4, KPAD),
                         lambda i: (i // (grid // 2), 0, 0)),
        ],
        out_specs=pl.BlockSpec((1, 1, K_MAX),
                               lambda i: (i // (grid // 2), 0, 0)),
        out_shape=jax.ShapeDtypeStruct((2, 1, K_MAX), jnp.float32),
    )(cpad2, stats)[:, 0, :]

    nk = stats[:, 0, :K_MAX]
    att_s = attrep[:, 0, :K_MAX]
    rep_o = attrep[:, 1, :K_MAX]
    exists = (nk > 0).astype(jnp.float32)
    c2 = jnp.float32(c_q * c_q)
    inv = c_q / jnp.maximum(nk * c_q, 1e-6)
    mn = jnp.sum((stats[:, 1:4, :K_MAX] * inv[:, None, :]) ** 2, axis=1)
    repall = repall - float(pad) * jnp.maximum(
        1.0 - jnp.sqrt(mn + 1e-9), 0.0)
    att = c2 * att_s / jnp.maximum(nk, 1.0)
    rep = c2 * (repall - rep_o) / jnp.maximum(float(n_ev) - nk, 1.0)
    n_obj = jnp.maximum(jnp.sum(exists, axis=1), 1.0)
    v_att = jnp.sum(att * exists, axis=1) / n_obj
    v_rep = jnp.sum(rep * exists, axis=1) / n_obj
    return jnp.sum(v_att + v_rep) / 2.0
